# baseline (speedup 1.0000x reference)
import functools

import jax
import jax.numpy as jnp
from jax import lax
from jax.experimental import pallas as pl
from jax.experimental.pallas import tpu as pltpu
from jax.experimental.pallas import tpu_sc as plsc

MAXLEN = 200
VOCAB_SIZE = 100000
EMBED_DIM = 64
BATCH = 4096

NC = 2
NS = 16
LANES = 16
NW = NC * NS
BW = BATCH // NW
Lc = 2
NCH = MAXLEN // Lc
DT = EMBED_DIM // 8
GB = BW // LANES

_mesh = plsc.VectorSubcoreMesh(core_axis_name="c", subcore_axis_name="s")


@functools.partial(
    pl.kernel,
    mesh=_mesh,
    out_type=jax.ShapeDtypeStruct((MAXLEN * 8, NW, 8, BW), jnp.float32),
    scratch_types=[
        pltpu.VMEM((Lc, BW), jnp.int32),
        pltpu.VMEM((Lc, BW), jnp.int32),
        pltpu.VMEM((Lc * BW, EMBED_DIM), jnp.float32),
        pltpu.VMEM((Lc * BW, EMBED_DIM), jnp.float32),
        pltpu.VMEM((Lc * 8, 1, 8, BW), jnp.float32),
        pltpu.VMEM((Lc * 8, 1, 8, BW), jnp.float32),
        pltpu.VMEM((MAXLEN, EMBED_DIM), jnp.float32),
        pltpu.SemaphoreType.DMA,
        pltpu.SemaphoreType.DMA,
        pltpu.SemaphoreType.DMA,
        pltpu.SemaphoreType.DMA,
        pltpu.SemaphoreType.DMA,
        pltpu.SemaphoreType.DMA,
    ],
    compiler_params=pltpu.CompilerParams(
        use_tc_tiling_on_sc=False, needs_layout_passes=False),
)
def _embed(idxT_hbm, tok_hbm, pos_hbm, out_hbm,
           idx0, idx1, rows0, rows1, tout0, tout1, pos_v,
           i0, i1, g0, g1, s0, s1):
    wid = lax.axis_index("s") * NC + lax.axis_index("c")
    b0 = wid * BW

    idx_bufs = (idx0, idx1)
    rows_bufs = (rows0, rows1)
    tout_bufs = (tout0, tout1)
    isems = (i0, i1)
    gsems = (g0, g1)
    ssems = (s0, s1)

    pltpu.sync_copy(pos_hbm, pos_v)

    iota = lax.iota(jnp.int32, LANES)
    bvecs = [[iota + (l_loc * BW + g * LANES) for g in range(GB)]
             for l_loc in range(Lc)]

    def stage_idx(c, bi):
        pltpu.async_copy(
            idxT_hbm.at[pl.ds(c * Lc, Lc), pl.ds(b0, BW)],
            idx_bufs[bi], isems[bi])

    def wait_idx(bi):
        pltpu.make_async_copy(
            idxT_hbm.at[pl.ds(0, Lc), pl.ds(0, BW)],
            idx_bufs[bi], isems[bi]).wait()

    def fire_gathers(bi):
        for l_loc in range(Lc):
            pltpu.async_copy(
                tok_hbm.at[idx_bufs[bi].at[l_loc]],
                rows_bufs[bi].at[pl.ds(l_loc * BW, BW)],
                gsems[bi])

    def wait_gathers(bi):
        pltpu.make_async_copy(
            tok_hbm.at[pl.ds(0, Lc * BW)], rows_bufs[bi], gsems[bi]).wait()

    def fire_scatter(c, bi):
        pltpu.async_copy(
            tout_bufs[bi],
            out_hbm.at[pl.ds(c * Lc * 8, Lc * 8), pl.ds(wid, 1)],
            ssems[bi])

    def wait_scatter(bi):
        pltpu.make_async_copy(
            tout_bufs[bi],
            out_hbm.at[pl.ds(0, Lc * 8), pl.ds(0, 1)], ssems[bi]).wait()

    def transpose_add(c, bi):
        rows = rows_bufs[bi]
        tout = tout_bufs[bi]
        for l_loc in range(Lc):
            lrow = c * Lc + l_loc
            lvec = jnp.full((LANES,), lrow, jnp.int32)
            bv = bvecs[l_loc]

            def dbody(d, carry):
                dt = d >> 3
                dr = d & 7
                dvec = jnp.full((LANES,), d, jnp.int32)
                ps = plsc.load_gather(pos_v, [lvec, dvec])
                for g in range(GB):
                    v = plsc.load_gather(rows, [bv[g], dvec])
                    tout[l_loc * 8 + dt, 0, dr, pl.ds(g * LANES, LANES)] = (
                        v + ps)
                return carry

            lax.fori_loop(0, EMBED_DIM, dbody, 0)

    stage_idx(0, 0)
    wait_idx(0)
    fire_gathers(0)
    stage_idx(1, 1)

    wait_gathers(0)
    wait_idx(1)
    fire_gathers(1)
    stage_idx(2, 0)
    transpose_add(0, 0)
    fire_scatter(0, 0)

    wait_gathers(1)
    wait_idx(0)
    fire_gathers(0)
    stage_idx(3, 1)
    transpose_add(1, 1)
    fire_scatter(1, 1)

    def pair_body(g, carry):
        for off in (0, 1):
            c = 2 * g + 2 + off
            bi = off
            nb = 1 - bi
            wait_gathers(bi)
            wait_idx(nb)
            fire_gathers(nb)
            stage_idx(c + 2, bi)
            wait_scatter(bi)
            transpose_add(c, bi)
            fire_scatter(c, bi)
        return carry

    lax.fori_loop(0, (NCH - 4) // 2, pair_body, 0)

    wait_gathers(0)
    wait_idx(1)
    fire_gathers(1)
    wait_scatter(0)
    transpose_add(NCH - 2, 0)
    fire_scatter(NCH - 2, 0)

    wait_gathers(1)
    wait_scatter(1)
    transpose_add(NCH - 1, 1)
    fire_scatter(NCH - 1, 1)

    wait_scatter(0)
    wait_scatter(1)


def kernel(inputs, token_table, pos_table):
    idxT = inputs.T.astype(jnp.int32)
    out4 = _embed(idxT, token_table, pos_table)
    t = out4.reshape(MAXLEN, 8, NW, 8, BW).transpose(0, 1, 3, 2, 4)
    return t.reshape(MAXLEN, EMBED_DIM, BATCH).transpose(2, 0, 1)


# speedup vs baseline: 6.2188x; 6.2188x over previous
import functools

import jax
import jax.numpy as jnp
from jax import lax
from jax.experimental import pallas as pl
from jax.experimental.pallas import tpu as pltpu
from jax.experimental.pallas import tpu_sc as plsc

MAXLEN = 200
VOCAB_SIZE = 100000
EMBED_DIM = 64
BATCH = 4096

NC = 2
NS = 16
LANES = 16
NW = NC * NS
BW = BATCH // NW
Lc = 4
Sc = 2
NCH = MAXLEN // Lc
GB = BW // LANES
BP = BW + 1

_mesh = plsc.VectorSubcoreMesh(core_axis_name="c", subcore_axis_name="s")


@functools.partial(
    pl.kernel,
    mesh=_mesh,
    out_type=jax.ShapeDtypeStruct((MAXLEN * 8, NW, 8, BW), jnp.float32),
    scratch_types=[
        pltpu.VMEM((Lc, BW), jnp.int32),
        pltpu.VMEM((Lc, BW), jnp.int32),
        pltpu.VMEM((Lc * BW, EMBED_DIM), jnp.float32),
        pltpu.VMEM((Lc * BW, EMBED_DIM), jnp.float32),
        pltpu.VMEM((Sc * 8, 1, 8, BP), jnp.float32),
        pltpu.VMEM((Sc * 8, 1, 8, BP), jnp.float32),
        pltpu.VMEM((MAXLEN, EMBED_DIM), jnp.float32),
        pltpu.SemaphoreType.DMA,
        pltpu.SemaphoreType.DMA,
        pltpu.SemaphoreType.DMA,
        pltpu.SemaphoreType.DMA,
        pltpu.SemaphoreType.DMA,
        pltpu.SemaphoreType.DMA,
    ],
    compiler_params=pltpu.CompilerParams(
        use_tc_tiling_on_sc=False, needs_layout_passes=False),
)
def _embed(idxT_hbm, tok_hbm, pos_hbm, out_hbm,
           idx0, idx1, rows0, rows1, tout0, tout1, pos_v,
           i0, i1, g0, g1, s0, s1):
    wid = lax.axis_index("s") * NC + lax.axis_index("c")
    b0 = wid * BW

    idx_bufs = (idx0, idx1)
    rows_bufs = (rows0, rows1)
    tout_bufs = (tout0, tout1)
    isems = (i0, i1)
    gsems = (g0, g1)
    ssems = (s0, s1)

    pltpu.sync_copy(pos_hbm, pos_v)

    iota = lax.iota(jnp.int32, LANES)

    def stage_idx(c, bi):
        pltpu.async_copy(
            idxT_hbm.at[pl.ds(c * Lc, Lc), pl.ds(b0, BW)],
            idx_bufs[bi], isems[bi])

    def wait_idx(bi):
        pltpu.make_async_copy(
            idxT_hbm.at[pl.ds(0, Lc), pl.ds(0, BW)],
            idx_bufs[bi], isems[bi]).wait()

    def fire_gathers(bi):
        for l_loc in range(Lc):
            pltpu.async_copy(
                tok_hbm.at[idx_bufs[bi].at[l_loc]],
                rows_bufs[bi].at[pl.ds(l_loc * BW, BW)],
                gsems[bi])

    def wait_gathers(bi):
        pltpu.make_async_copy(
            tok_hbm.at[pl.ds(0, Lc * BW)], rows_bufs[bi], gsems[bi]).wait()

    def fire_scatter(c, h):
        pltpu.async_copy(
            tout_bufs[h].at[:, :, :, pl.ds(0, BW)],
            out_hbm.at[pl.ds((c * Lc + h * Sc) * 8, Sc * 8), pl.ds(wid, 1)],
            ssems[h])

    def wait_scatter(h):
        pltpu.make_async_copy(
            tout_bufs[h].at[:, :, :, pl.ds(0, BW)],
            out_hbm.at[pl.ds(0, Sc * 8), pl.ds(0, 1)], ssems[h]).wait()

    def transpose_add(c, bi, h):
        rows = rows_bufs[bi]
        tout = tout_bufs[h]
        for sl in range(Sc):
            l_loc = h * Sc + sl
            lrow = c * Lc + l_loc
            pos_q = [pos_v[lrow, pl.ds(q * LANES, LANES)] for q in range(4)]
            av = [((sl * 8 + 2 * q) + (iota >> 3)).astype(jnp.int32)
                  for q in range(4)]
            drv = iota & 7
            zv = jnp.zeros((LANES,), jnp.int32)

            @plsc.parallel_loop(0, BW, 1, unroll=4)
            def _bbody(b):
                row = l_loc * BW + b
                bsp = jnp.full((LANES,), b, jnp.int32)
                for q in range(4):
                    v = rows[row, pl.ds(q * LANES, LANES)] + pos_q[q]
                    plsc.store_scatter(tout, [av[q], zv, drv, bsp], v)

    def process(c, bi, first):
        for h in (0, 1):
            if not first:
                wait_scatter(h)
            transpose_add(c, bi, h)
            fire_scatter(c, h)

    stage_idx(0, 0)
    wait_idx(0)
    fire_gathers(0)
    stage_idx(1, 1)

    wait_gathers(0)
    wait_idx(1)
    fire_gathers(1)
    stage_idx(2, 0)
    process(0, 0, True)

    wait_gathers(1)
    wait_idx(0)
    fire_gathers(0)
    stage_idx(3, 1)
    process(1, 1, False)

    def pair_body(g, carry):
        for off in (0, 1):
            c = 2 * g + 2 + off
            bi = off
            nb = 1 - bi
            wait_gathers(bi)
            wait_idx(nb)
            fire_gathers(nb)
            stage_idx(c + 2, bi)
            process(c, bi, False)
        return carry

    lax.fori_loop(0, (NCH - 4) // 2, pair_body, 0)

    wait_gathers(0)
    wait_idx(1)
    fire_gathers(1)
    process(NCH - 2, 0, False)

    wait_gathers(1)
    process(NCH - 1, 1, False)

    wait_scatter(0)
    wait_scatter(1)


def kernel(inputs, token_table, pos_table):
    idxT = inputs.T.astype(jnp.int32)
    out4 = _embed(idxT, token_table, pos_table)
    t = out4.reshape(MAXLEN, 8, NW, 8, BW).transpose(0, 1, 3, 2, 4)
    return t.reshape(MAXLEN, EMBED_DIM, BATCH).transpose(2, 0, 1)


# speedup vs baseline: 6.4799x; 1.0420x over previous
import functools

import jax
import jax.numpy as jnp
from jax import lax
from jax.experimental import pallas as pl
from jax.experimental.pallas import tpu as pltpu
from jax.experimental.pallas import tpu_sc as plsc

MAXLEN = 200
VOCAB_SIZE = 100000
EMBED_DIM = 64
BATCH = 4096

NC = 2
NS = 16
LANES = 16
NW = NC * NS
BW = BATCH // NW
Lc = 4
Sc = 2
NCH = MAXLEN // Lc
GB = BW // LANES
BP = BW + 1

_mesh = plsc.VectorSubcoreMesh(core_axis_name="c", subcore_axis_name="s")

TTB = 782
TPW = 24
OBP = 65


@functools.partial(
    pl.kernel,
    mesh=_mesh,
    out_type=jax.ShapeDtypeStruct((VOCAB_SIZE, EMBED_DIM), jnp.float32),
    scratch_types=[
        pltpu.VMEM((8, 1, 8, 128), jnp.float32),
        pltpu.VMEM((8, 1, 8, 128), jnp.float32),
        pltpu.VMEM((128, OBP), jnp.float32),
        pltpu.VMEM((128, OBP), jnp.float32),
        pltpu.SemaphoreType.DMA,
        pltpu.SemaphoreType.DMA,
        pltpu.SemaphoreType.DMA,
        pltpu.SemaphoreType.DMA,
    ],
    compiler_params=pltpu.CompilerParams(
        use_tc_tiling_on_sc=False, needs_layout_passes=False),
)
def _detile(tok4, out, blk0, blk1, ob0, ob1, r0, r1, w0, w1):
    wid = lax.axis_index("s") * NC + lax.axis_index("c")
    start = wid * TPW

    blks = (blk0, blk1)
    obs = (ob0, ob1)
    rsems = (r0, r1)
    wsems = (w0, w1)

    iota = lax.iota(jnp.int32, LANES)
    rvecs = [iota + tig * LANES for tig in range(8)]

    def read_blk(tt, bi):
        pltpu.async_copy(tok4.at[:, pl.ds(tt, 1)], blks[bi], rsems[bi])

    def wait_read(bi):
        pltpu.make_async_copy(
            tok4.at[:, pl.ds(0, 1)], blks[bi], rsems[bi]).wait()

    def write_ob(tt, bi, nrows):
        pltpu.async_copy(
            obs[bi].at[pl.ds(0, nrows), pl.ds(0, EMBED_DIM)],
            out.at[pl.ds(tt * 128, nrows)], wsems[bi])

    def wait_write(bi, nrows):
        pltpu.make_async_copy(
            obs[bi].at[pl.ds(0, nrows), pl.ds(0, EMBED_DIM)],
            out.at[pl.ds(0, nrows)], wsems[bi]).wait()

    def transpose_blk(bi):
        blk = blks[bi]
        ob = obs[bi]

        @plsc.parallel_loop(0, EMBED_DIM, 1, unroll=2)
        def _dbody(d):
            dt = d >> 3
            di = d & 7
            dsp = jnp.full((LANES,), d, jnp.int32)
            for tig in range(8):
                v = blk[dt, 0, di, pl.ds(tig * LANES, LANES)]
                plsc.store_scatter(ob, [rvecs[tig], dsp], v)

    read_blk(start, 0)
    read_blk(start + 1, 1)

    def pair_body(g, carry):
        for bi in (0, 1):
            i = 2 * g + bi
            wait_read(bi)
            transpose_blk(bi)

            @pl.when(i >= 2)
            def _():
                wait_write(bi, 128)

            write_ob(start + i, bi, 128)

            @pl.when(i + 2 < TPW)
            def _():
                read_blk(start + i + 2, bi)

        return carry

    lax.fori_loop(0, TPW // 2, pair_body, 0)
    wait_write(0, 128)
    wait_write(1, 128)

    @pl.when(wid <= 12)
    def _():
        tt = 32 * TPW + wid
        read_blk(tt, 0)
        wait_read(0)
        transpose_blk(0)
        write_ob(tt, 0, 128)
        wait_write(0, 128)

    @pl.when(wid == 13)
    def _():
        tt = TTB - 1
        read_blk(tt, 0)
        wait_read(0)
        transpose_blk(0)
        write_ob(tt, 0, VOCAB_SIZE - (TTB - 1) * 128)
        wait_write(0, VOCAB_SIZE - (TTB - 1) * 128)


@functools.partial(
    pl.kernel,
    mesh=_mesh,
    out_type=jax.ShapeDtypeStruct((MAXLEN * 8, NW, 8, BW), jnp.float32),
    scratch_types=[
        pltpu.VMEM((Lc, BW), jnp.int32),
        pltpu.VMEM((Lc, BW), jnp.int32),
        pltpu.VMEM((Lc * BW, EMBED_DIM), jnp.float32),
        pltpu.VMEM((Lc * BW, EMBED_DIM), jnp.float32),
        pltpu.VMEM((Sc * 8, 1, 8, BP), jnp.float32),
        pltpu.VMEM((Sc * 8, 1, 8, BP), jnp.float32),
        pltpu.VMEM((MAXLEN, EMBED_DIM), jnp.float32),
        pltpu.SemaphoreType.DMA,
        pltpu.SemaphoreType.DMA,
        pltpu.SemaphoreType.DMA,
        pltpu.SemaphoreType.DMA,
        pltpu.SemaphoreType.DMA,
        pltpu.SemaphoreType.DMA,
    ],
    compiler_params=pltpu.CompilerParams(
        use_tc_tiling_on_sc=False, needs_layout_passes=False),
)
def _embed(idxT_hbm, tok_hbm, pos_hbm, out_hbm,
           idx0, idx1, rows0, rows1, tout0, tout1, pos_v,
           i0, i1, g0, g1, s0, s1):
    wid = lax.axis_index("s") * NC + lax.axis_index("c")
    b0 = wid * BW

    idx_bufs = (idx0, idx1)
    rows_bufs = (rows0, rows1)
    tout_bufs = (tout0, tout1)
    isems = (i0, i1)
    gsems = (g0, g1)
    ssems = (s0, s1)

    pltpu.sync_copy(pos_hbm, pos_v)

    iota = lax.iota(jnp.int32, LANES)

    def stage_idx(c, bi):
        pltpu.async_copy(
            idxT_hbm.at[pl.ds(c * Lc, Lc), pl.ds(b0, BW)],
            idx_bufs[bi], isems[bi])

    def wait_idx(bi):
        pltpu.make_async_copy(
            idxT_hbm.at[pl.ds(0, Lc), pl.ds(0, BW)],
            idx_bufs[bi], isems[bi]).wait()

    def fire_gathers(bi):
        for l_loc in range(Lc):
            pltpu.async_copy(
                tok_hbm.at[idx_bufs[bi].at[l_loc]],
                rows_bufs[bi].at[pl.ds(l_loc * BW, BW)],
                gsems[bi])

    def wait_gathers(bi):
        pltpu.make_async_copy(
            tok_hbm.at[pl.ds(0, Lc * BW)], rows_bufs[bi], gsems[bi]).wait()

    def fire_scatter(c, h):
        pltpu.async_copy(
            tout_bufs[h].at[:, :, :, pl.ds(0, BW)],
            out_hbm.at[pl.ds((c * Lc + h * Sc) * 8, Sc * 8), pl.ds(wid, 1)],
            ssems[h])

    def wait_scatter(h):
        pltpu.make_async_copy(
            tout_bufs[h].at[:, :, :, pl.ds(0, BW)],
            out_hbm.at[pl.ds(0, Sc * 8), pl.ds(0, 1)], ssems[h]).wait()

    def transpose_add(c, bi, h):
        rows = rows_bufs[bi]
        tout = tout_bufs[h]
        for sl in range(Sc):
            l_loc = h * Sc + sl
            lrow = c * Lc + l_loc
            pos_q = [pos_v[lrow, pl.ds(q * LANES, LANES)] for q in range(4)]
            av = [((sl * 8 + 2 * q) + (iota >> 3)).astype(jnp.int32)
                  for q in range(4)]
            drv = iota & 7
            zv = jnp.zeros((LANES,), jnp.int32)

            @plsc.parallel_loop(0, BW, 1, unroll=4)
            def _bbody(b):
                row = l_loc * BW + b
                bsp = jnp.full((LANES,), b, jnp.int32)
                for q in range(4):
                    v = rows[row, pl.ds(q * LANES, LANES)] + pos_q[q]
                    plsc.store_scatter(tout, [av[q], zv, drv, bsp], v)

    def process(c, bi, first):
        for h in (0, 1):
            if not first:
                wait_scatter(h)
            transpose_add(c, bi, h)
            fire_scatter(c, h)

    stage_idx(0, 0)
    wait_idx(0)
    fire_gathers(0)
    stage_idx(1, 1)

    wait_gathers(0)
    wait_idx(1)
    fire_gathers(1)
    stage_idx(2, 0)
    process(0, 0, True)

    wait_gathers(1)
    wait_idx(0)
    fire_gathers(0)
    stage_idx(3, 1)
    process(1, 1, False)

    def pair_body(g, carry):
        for off in (0, 1):
            c = 2 * g + 2 + off
            bi = off
            nb = 1 - bi
            wait_gathers(bi)
            wait_idx(nb)
            fire_gathers(nb)
            stage_idx(c + 2, bi)
            process(c, bi, False)
        return carry

    lax.fori_loop(0, (NCH - 4) // 2, pair_body, 0)

    wait_gathers(0)
    wait_idx(1)
    fire_gathers(1)
    process(NCH - 2, 0, False)

    wait_gathers(1)
    process(NCH - 1, 1, False)

    wait_scatter(0)
    wait_scatter(1)


def kernel(inputs, token_table, pos_table):
    idxT = inputs.T.astype(jnp.int32)
    tok4 = jnp.pad(token_table.T, ((0, 0), (0, TTB * 128 - VOCAB_SIZE)))
    tok4 = tok4.reshape(8, 8, TTB, 128).transpose(0, 2, 1, 3)
    tokrm = _detile(tok4)
    out4 = _embed(idxT, tokrm, pos_table)
    t = out4.reshape(MAXLEN, 8, NW, 8, BW).transpose(0, 1, 3, 2, 4)
    return t.reshape(MAXLEN, EMBED_DIM, BATCH).transpose(2, 0, 1)


# speedup vs baseline: 6.5026x; 1.0035x over previous
import functools

import jax
import jax.numpy as jnp
from jax import lax
from jax.experimental import pallas as pl
from jax.experimental.pallas import tpu as pltpu
from jax.experimental.pallas import tpu_sc as plsc

MAXLEN = 200
VOCAB_SIZE = 100000
EMBED_DIM = 64
BATCH = 4096

NC = 2
NS = 16
LANES = 16
NW = NC * NS
BW = BATCH // NW
Lc = 4
Sc = 2
NCH = MAXLEN // Lc
GB = BW // LANES
BP = BW + 1

_mesh = plsc.VectorSubcoreMesh(core_axis_name="c", subcore_axis_name="s")

TTB = 782
TPW = 24
OBP = 65


@functools.partial(
    pl.kernel,
    mesh=_mesh,
    out_type=jax.ShapeDtypeStruct((VOCAB_SIZE, EMBED_DIM), jnp.float32),
    scratch_types=[
        pltpu.VMEM((8, 1, 8, 128), jnp.float32),
        pltpu.VMEM((8, 1, 8, 128), jnp.float32),
        pltpu.VMEM((128, OBP), jnp.float32),
        pltpu.VMEM((128, OBP), jnp.float32),
        pltpu.SemaphoreType.DMA,
        pltpu.SemaphoreType.DMA,
        pltpu.SemaphoreType.DMA,
        pltpu.SemaphoreType.DMA,
    ],
    compiler_params=pltpu.CompilerParams(
        use_tc_tiling_on_sc=False, needs_layout_passes=False),
)
def _detile(tok4, out, blk0, blk1, ob0, ob1, r0, r1, w0, w1):
    wid = lax.axis_index("s") * NC + lax.axis_index("c")
    start = wid * TPW

    blks = (blk0, blk1)
    obs = (ob0, ob1)
    rsems = (r0, r1)
    wsems = (w0, w1)

    iota = lax.iota(jnp.int32, LANES)
    rvecs = [iota + tig * LANES for tig in range(8)]

    def read_blk(tt, bi):
        pltpu.async_copy(tok4.at[:, pl.ds(tt, 1)], blks[bi], rsems[bi])

    def wait_read(bi):
        pltpu.make_async_copy(
            tok4.at[:, pl.ds(0, 1)], blks[bi], rsems[bi]).wait()

    def write_ob(tt, bi, nrows):
        pltpu.async_copy(
            obs[bi].at[pl.ds(0, nrows), pl.ds(0, EMBED_DIM)],
            out.at[pl.ds(tt * 128, nrows)], wsems[bi])

    def wait_write(bi, nrows):
        pltpu.make_async_copy(
            obs[bi].at[pl.ds(0, nrows), pl.ds(0, EMBED_DIM)],
            out.at[pl.ds(0, nrows)], wsems[bi]).wait()

    def transpose_blk(bi):
        blk = blks[bi]
        ob = obs[bi]

        @plsc.parallel_loop(0, EMBED_DIM, 1, unroll=2)
        def _dbody(d):
            dt = d >> 3
            di = d & 7
            dsp = jnp.full((LANES,), d, jnp.int32)
            for tig in range(8):
                v = blk[dt, 0, di, pl.ds(tig * LANES, LANES)]
                plsc.store_scatter(ob, [rvecs[tig], dsp], v)

    read_blk(start, 0)
    read_blk(start + 1, 1)

    def pair_body(g, carry):
        for bi in (0, 1):
            i = 2 * g + bi
            wait_read(bi)

            @pl.when(i >= 2)
            def _():
                wait_write(bi, 128)

            transpose_blk(bi)
            write_ob(start + i, bi, 128)

            @pl.when(i + 2 < TPW)
            def _():
                read_blk(start + i + 2, bi)

        return carry

    lax.fori_loop(0, TPW // 2, pair_body, 0)
    wait_write(0, 128)
    wait_write(1, 128)

    @pl.when(wid <= 12)
    def _():
        tt = 32 * TPW + wid
        read_blk(tt, 0)
        wait_read(0)
        transpose_blk(0)
        write_ob(tt, 0, 128)
        wait_write(0, 128)

    @pl.when(wid == 13)
    def _():
        tt = TTB - 1
        read_blk(tt, 0)
        wait_read(0)
        transpose_blk(0)
        write_ob(tt, 0, VOCAB_SIZE - (TTB - 1) * 128)
        wait_write(0, VOCAB_SIZE - (TTB - 1) * 128)


@functools.partial(
    pl.kernel,
    mesh=_mesh,
    out_type=jax.ShapeDtypeStruct((MAXLEN * 8, NW, 8, BW), jnp.float32),
    scratch_types=[
        pltpu.VMEM((Lc, BW), jnp.int32),
        pltpu.VMEM((Lc, BW), jnp.int32),
        pltpu.VMEM((Lc * BW, EMBED_DIM), jnp.float32),
        pltpu.VMEM((Lc * BW, EMBED_DIM), jnp.float32),
        pltpu.VMEM((Sc * 8, 1, 8, BP), jnp.float32),
        pltpu.VMEM((Sc * 8, 1, 8, BP), jnp.float32),
        pltpu.VMEM((MAXLEN, EMBED_DIM), jnp.float32),
        pltpu.SemaphoreType.DMA,
        pltpu.SemaphoreType.DMA,
        pltpu.SemaphoreType.DMA,
        pltpu.SemaphoreType.DMA,
        pltpu.SemaphoreType.DMA,
        pltpu.SemaphoreType.DMA,
    ],
    compiler_params=pltpu.CompilerParams(
        use_tc_tiling_on_sc=False, needs_layout_passes=False),
)
def _embed(idxT_hbm, tok_hbm, pos_hbm, out_hbm,
           idx0, idx1, rows0, rows1, tout0, tout1, pos_v,
           i0, i1, g0, g1, s0, s1):
    wid = lax.axis_index("s") * NC + lax.axis_index("c")
    b0 = wid * BW

    idx_bufs = (idx0, idx1)
    rows_bufs = (rows0, rows1)
    tout_bufs = (tout0, tout1)
    isems = (i0, i1)
    gsems = (g0, g1)
    ssems = (s0, s1)

    pltpu.sync_copy(pos_hbm, pos_v)

    iota = lax.iota(jnp.int32, LANES)

    def stage_idx(c, bi):
        pltpu.async_copy(
            idxT_hbm.at[pl.ds(c * Lc, Lc), pl.ds(b0, BW)],
            idx_bufs[bi], isems[bi])

    def wait_idx(bi):
        pltpu.make_async_copy(
            idxT_hbm.at[pl.ds(0, Lc), pl.ds(0, BW)],
            idx_bufs[bi], isems[bi]).wait()

    def fire_gathers(bi):
        for l_loc in range(Lc):
            pltpu.async_copy(
                tok_hbm.at[idx_bufs[bi].at[l_loc]],
                rows_bufs[bi].at[pl.ds(l_loc * BW, BW)],
                gsems[bi])

    def wait_gathers(bi):
        pltpu.make_async_copy(
            tok_hbm.at[pl.ds(0, Lc * BW)], rows_bufs[bi], gsems[bi]).wait()

    def fire_scatter(c, h):
        pltpu.async_copy(
            tout_bufs[h].at[:, :, :, pl.ds(0, BW)],
            out_hbm.at[pl.ds((c * Lc + h * Sc) * 8, Sc * 8), pl.ds(wid, 1)],
            ssems[h])

    def wait_scatter(h):
        pltpu.make_async_copy(
            tout_bufs[h].at[:, :, :, pl.ds(0, BW)],
            out_hbm.at[pl.ds(0, Sc * 8), pl.ds(0, 1)], ssems[h]).wait()

    def transpose_add(c, bi, h):
        rows = rows_bufs[bi]
        tout = tout_bufs[h]
        for sl in range(Sc):
            l_loc = h * Sc + sl
            lrow = c * Lc + l_loc
            pos_q = [pos_v[lrow, pl.ds(q * LANES, LANES)] for q in range(4)]
            av = [((sl * 8 + 2 * q) + (iota >> 3)).astype(jnp.int32)
                  for q in range(4)]
            drv = iota & 7
            zv = jnp.zeros((LANES,), jnp.int32)

            @plsc.parallel_loop(0, BW, 1, unroll=4)
            def _bbody(b):
                row = l_loc * BW + b
                bsp = jnp.full((LANES,), b, jnp.int32)
                for q in range(4):
                    v = rows[row, pl.ds(q * LANES, LANES)] + pos_q[q]
                    plsc.store_scatter(tout, [av[q], zv, drv, bsp], v)

    def process(c, bi, first):
        for h in (0, 1):
            if not first:
                wait_scatter(h)
            transpose_add(c, bi, h)
            fire_scatter(c, h)

    stage_idx(0, 0)
    wait_idx(0)
    fire_gathers(0)
    stage_idx(1, 1)

    wait_gathers(0)
    wait_idx(1)
    fire_gathers(1)
    stage_idx(2, 0)
    process(0, 0, True)

    wait_gathers(1)
    wait_idx(0)
    fire_gathers(0)
    stage_idx(3, 1)
    process(1, 1, False)

    def pair_body(g, carry):
        for off in (0, 1):
            c = 2 * g + 2 + off
            bi = off
            nb = 1 - bi
            wait_gathers(bi)
            wait_idx(nb)
            fire_gathers(nb)
            stage_idx(c + 2, bi)
            process(c, bi, False)
        return carry

    lax.fori_loop(0, (NCH - 4) // 2, pair_body, 0)

    wait_gathers(0)
    wait_idx(1)
    fire_gathers(1)
    process(NCH - 2, 0, False)

    wait_gathers(1)
    process(NCH - 1, 1, False)

    wait_scatter(0)
    wait_scatter(1)


def kernel(inputs, token_table, pos_table):
    idxT = inputs.T.astype(jnp.int32)
    tok4 = jnp.pad(token_table.T, ((0, 0), (0, TTB * 128 - VOCAB_SIZE)))
    tok4 = tok4.reshape(8, 8, TTB, 128).transpose(0, 2, 1, 3)
    tokrm = _detile(tok4)
    out4 = _embed(idxT, tokrm, pos_table)
    t = out4.reshape(MAXLEN, 8, NW, 8, BW).transpose(0, 1, 3, 2, 4)
    return t.reshape(MAXLEN, EMBED_DIM, BATCH).transpose(2, 0, 1)


# speedup vs baseline: 6.5389x; 1.0056x over previous
import functools

import jax
import jax.numpy as jnp
from jax import lax
from jax.experimental import pallas as pl
from jax.experimental.pallas import tpu as pltpu
from jax.experimental.pallas import tpu_sc as plsc

MAXLEN = 200
VOCAB_SIZE = 100000
EMBED_DIM = 64
BATCH = 4096

NC = 2
NS = 16
LANES = 16
NW = NC * NS
BW = BATCH // NW
Lc = 4
Sc = 2
NCH = MAXLEN // Lc
GB = BW // LANES
BP = BW + 1

_mesh = plsc.VectorSubcoreMesh(core_axis_name="c", subcore_axis_name="s")

TTB = 782
TPW = 24
OBP = 65


@functools.partial(
    pl.kernel,
    mesh=_mesh,
    out_type=jax.ShapeDtypeStruct((VOCAB_SIZE, EMBED_DIM), jnp.float32),
    scratch_types=[
        pltpu.VMEM((8, 1, 8, 128), jnp.float32),
        pltpu.VMEM((8, 1, 8, 128), jnp.float32),
        pltpu.VMEM((128, OBP), jnp.float32),
        pltpu.VMEM((128, OBP), jnp.float32),
        pltpu.SemaphoreType.DMA,
        pltpu.SemaphoreType.DMA,
        pltpu.SemaphoreType.DMA,
        pltpu.SemaphoreType.DMA,
    ],
    compiler_params=pltpu.CompilerParams(
        use_tc_tiling_on_sc=False, needs_layout_passes=False),
)
def _detile(tok4, out, blk0, blk1, ob0, ob1, r0, r1, w0, w1):
    wid = lax.axis_index("s") * NC + lax.axis_index("c")
    start = wid * TPW

    blks = (blk0, blk1)
    obs = (ob0, ob1)
    rsems = (r0, r1)
    wsems = (w0, w1)

    iota = lax.iota(jnp.int32, LANES)
    rvecs = [iota + tig * LANES for tig in range(8)]

    def read_blk(tt, bi):
        pltpu.async_copy(tok4.at[:, pl.ds(tt, 1)], blks[bi], rsems[bi])

    def wait_read(bi):
        pltpu.make_async_copy(
            tok4.at[:, pl.ds(0, 1)], blks[bi], rsems[bi]).wait()

    def write_ob(tt, bi, nrows):
        pltpu.async_copy(
            obs[bi].at[pl.ds(0, nrows), pl.ds(0, EMBED_DIM)],
            out.at[pl.ds(tt * 128, nrows)], wsems[bi])

    def wait_write(bi, nrows):
        pltpu.make_async_copy(
            obs[bi].at[pl.ds(0, nrows), pl.ds(0, EMBED_DIM)],
            out.at[pl.ds(0, nrows)], wsems[bi]).wait()

    def transpose_blk(bi):
        blk = blks[bi]
        ob = obs[bi]

        @plsc.parallel_loop(0, EMBED_DIM, 1, unroll=2)
        def _dbody(d):
            dt = d >> 3
            di = d & 7
            dsp = jnp.full((LANES,), d, jnp.int32)
            for tig in range(8):
                v = blk[dt, 0, di, pl.ds(tig * LANES, LANES)]
                plsc.store_scatter(ob, [rvecs[tig], dsp], v)

    read_blk(start, 0)
    read_blk(start + 1, 1)

    def pair_body(g, carry):
        for bi in (0, 1):
            i = 2 * g + bi
            wait_read(bi)

            @pl.when(i >= 2)
            def _():
                wait_write(bi, 128)

            transpose_blk(bi)
            write_ob(start + i, bi, 128)

            @pl.when(i + 2 < TPW)
            def _():
                read_blk(start + i + 2, bi)

        return carry

    lax.fori_loop(0, TPW // 2, pair_body, 0)
    wait_write(0, 128)
    wait_write(1, 128)

    @pl.when(wid <= 12)
    def _():
        tt = 32 * TPW + wid
        read_blk(tt, 0)
        wait_read(0)
        transpose_blk(0)
        write_ob(tt, 0, 128)
        wait_write(0, 128)

    @pl.when(wid == 13)
    def _():
        tt = TTB - 1
        read_blk(tt, 0)
        wait_read(0)
        transpose_blk(0)
        write_ob(tt, 0, VOCAB_SIZE - (TTB - 1) * 128)
        wait_write(0, VOCAB_SIZE - (TTB - 1) * 128)


@functools.partial(
    pl.kernel,
    mesh=_mesh,
    out_type=jax.ShapeDtypeStruct((MAXLEN * 8, NW, 8, BW), jnp.float32),
    scratch_types=[
        pltpu.VMEM((Lc, BW), jnp.int32),
        pltpu.VMEM((Lc, BW), jnp.int32),
        pltpu.VMEM((Lc * BW, EMBED_DIM), jnp.float32),
        pltpu.VMEM((Lc * BW, EMBED_DIM), jnp.float32),
        pltpu.VMEM((Sc * 8, 1, 8, BP), jnp.float32),
        pltpu.VMEM((Sc * 8, 1, 8, BP), jnp.float32),
        pltpu.VMEM((MAXLEN, EMBED_DIM), jnp.float32),
        pltpu.SemaphoreType.DMA,
        pltpu.SemaphoreType.DMA,
        pltpu.SemaphoreType.DMA,
        pltpu.SemaphoreType.DMA,
        pltpu.SemaphoreType.DMA,
        pltpu.SemaphoreType.DMA,
    ],
    compiler_params=pltpu.CompilerParams(
        use_tc_tiling_on_sc=False, needs_layout_passes=False),
)
def _embed(idxT_hbm, tok_hbm, pos_hbm, out_hbm,
           idx0, idx1, rows0, rows1, tout0, tout1, pos_v,
           i0, i1, g0, g1, s0, s1):
    wid = lax.axis_index("s") * NC + lax.axis_index("c")
    b0 = wid * BW

    idx_bufs = (idx0, idx1)
    rows_bufs = (rows0, rows1)
    tout_bufs = (tout0, tout1)
    isems = (i0, i1)
    gsems = (g0, g1)
    ssems = (s0, s1)

    pltpu.sync_copy(pos_hbm, pos_v)

    iota = lax.iota(jnp.int32, LANES)

    def stage_idx(c, bi):
        pltpu.async_copy(
            idxT_hbm.at[pl.ds(c * Lc, Lc), pl.ds(b0, BW)],
            idx_bufs[bi], isems[bi])

    def wait_idx(bi):
        pltpu.make_async_copy(
            idxT_hbm.at[pl.ds(0, Lc), pl.ds(0, BW)],
            idx_bufs[bi], isems[bi]).wait()

    def fire_gathers(bi):
        for l_loc in range(Lc):
            pltpu.async_copy(
                tok_hbm.at[idx_bufs[bi].at[l_loc]],
                rows_bufs[bi].at[pl.ds(l_loc * BW, BW)],
                gsems[bi])

    def wait_gathers(bi):
        pltpu.make_async_copy(
            tok_hbm.at[pl.ds(0, Lc * BW)], rows_bufs[bi], gsems[bi]).wait()

    def fire_scatter(c, h):
        pltpu.async_copy(
            tout_bufs[h].at[:, :, :, pl.ds(0, BW)],
            out_hbm.at[pl.ds((c * Lc + h * Sc) * 8, Sc * 8), pl.ds(wid, 1)],
            ssems[h])

    def wait_scatter(h):
        pltpu.make_async_copy(
            tout_bufs[h].at[:, :, :, pl.ds(0, BW)],
            out_hbm.at[pl.ds(0, Sc * 8), pl.ds(0, 1)], ssems[h]).wait()

    def transpose_add(c, bi, h):
        rows = rows_bufs[bi]
        tout = tout_bufs[h]
        for sl in range(Sc):
            l_loc = h * Sc + sl
            lrow = c * Lc + l_loc
            pos_q = [pos_v[lrow, pl.ds(q * LANES, LANES)] for q in range(4)]
            av = [((sl * 8 + 2 * q) + (iota >> 3)).astype(jnp.int32)
                  for q in range(4)]
            drv = iota & 7
            zv = jnp.zeros((LANES,), jnp.int32)

            @plsc.parallel_loop(0, BW, 1, unroll=4)
            def _bbody(b):
                row = l_loc * BW + b
                bsp = jnp.full((LANES,), b, jnp.int32)
                for q in range(4):
                    v = rows[row, pl.ds(q * LANES, LANES)] + pos_q[q]
                    plsc.store_scatter(tout, [av[q], zv, drv, bsp], v)

    def process(c, bi, first):
        for h in (0, 1):
            if not first:
                wait_scatter(h)
            transpose_add(c, bi, h)
            fire_scatter(c, h)

    stage_idx(0, 0)
    wait_idx(0)
    fire_gathers(0)
    stage_idx(1, 1)

    wait_idx(1)
    fire_gathers(1)
    wait_gathers(0)
    stage_idx(2, 0)
    process(0, 0, True)

    wait_idx(0)
    fire_gathers(0)
    wait_gathers(1)
    stage_idx(3, 1)
    process(1, 1, False)

    def pair_body(g, carry):
        for off in (0, 1):
            c = 2 * g + 2 + off
            bi = off
            nb = 1 - bi
            wait_idx(nb)
            fire_gathers(nb)
            wait_gathers(bi)
            stage_idx(c + 2, bi)
            process(c, bi, False)
        return carry

    lax.fori_loop(0, (NCH - 4) // 2, pair_body, 0)

    wait_idx(1)
    fire_gathers(1)
    wait_gathers(0)
    process(NCH - 2, 0, False)

    wait_gathers(1)
    process(NCH - 1, 1, False)

    wait_scatter(0)
    wait_scatter(1)


def kernel(inputs, token_table, pos_table):
    idxT = inputs.T.astype(jnp.int32)
    tok4 = jnp.pad(token_table.T, ((0, 0), (0, TTB * 128 - VOCAB_SIZE)))
    tok4 = tok4.reshape(8, 8, TTB, 128).transpose(0, 2, 1, 3)
    tokrm = _detile(tok4)
    out4 = _embed(idxT, tokrm, pos_table)
    t = out4.reshape(MAXLEN, 8, NW, 8, BW).transpose(0, 1, 3, 2, 4)
    return t.reshape(MAXLEN, EMBED_DIM, BATCH).transpose(2, 0, 1)


# speedup vs baseline: 6.5553x; 1.0025x over previous
import functools

import jax
import jax.numpy as jnp
from jax import lax
from jax.experimental import pallas as pl
from jax.experimental.pallas import tpu as pltpu
from jax.experimental.pallas import tpu_sc as plsc

MAXLEN = 200
VOCAB_SIZE = 100000
EMBED_DIM = 64
BATCH = 4096

NC = 2
NS = 16
LANES = 16
NW = NC * NS
BW = BATCH // NW
Lc = 4
Sc = 2
NCH = MAXLEN // Lc
GB = BW // LANES
BP = BW + 1

_mesh = plsc.VectorSubcoreMesh(core_axis_name="c", subcore_axis_name="s")

TTB = 782
TPW = 24
OBP = 65


@functools.partial(
    pl.kernel,
    mesh=_mesh,
    out_type=jax.ShapeDtypeStruct((VOCAB_SIZE, EMBED_DIM), jnp.float32),
    scratch_types=[
        pltpu.VMEM((8, 2, 8, 128), jnp.float32),
        pltpu.VMEM((8, 2, 8, 128), jnp.float32),
        pltpu.VMEM((256, OBP), jnp.float32),
        pltpu.VMEM((256, OBP), jnp.float32),
        pltpu.SemaphoreType.DMA,
        pltpu.SemaphoreType.DMA,
        pltpu.SemaphoreType.DMA,
        pltpu.SemaphoreType.DMA,
    ],
    compiler_params=pltpu.CompilerParams(
        use_tc_tiling_on_sc=False, needs_layout_passes=False),
)
def _detile(tok4, out, blk0, blk1, ob0, ob1, r0, r1, w0, w1):
    wid = lax.axis_index("s") * NC + lax.axis_index("c")
    start = wid * TPW

    blks = (blk0, blk1)
    obs = (ob0, ob1)
    rsems = (r0, r1)
    wsems = (w0, w1)

    iota = lax.iota(jnp.int32, LANES)
    rvecs = [[iota + (tp * 128 + tig * LANES) for tig in range(8)]
             for tp in range(2)]

    def read_blk(tt, bi):
        pltpu.async_copy(tok4.at[:, pl.ds(tt, 2)], blks[bi], rsems[bi])

    def wait_read(bi):
        pltpu.make_async_copy(
            tok4.at[:, pl.ds(0, 2)], blks[bi], rsems[bi]).wait()

    def write_ob(tt, bi, nrows):
        pltpu.async_copy(
            obs[bi].at[pl.ds(0, nrows), pl.ds(0, EMBED_DIM)],
            out.at[pl.ds(tt * 128, nrows)], wsems[bi])

    def wait_write(bi, nrows):
        pltpu.make_async_copy(
            obs[bi].at[pl.ds(0, nrows), pl.ds(0, EMBED_DIM)],
            out.at[pl.ds(0, nrows)], wsems[bi]).wait()

    def transpose_blk(bi):
        blk = blks[bi]
        ob = obs[bi]

        @plsc.parallel_loop(0, EMBED_DIM, 1, unroll=2)
        def _dbody(d):
            dt = d >> 3
            di = d & 7
            dsp = jnp.full((LANES,), d, jnp.int32)
            for tp in range(2):
                for tig in range(8):
                    v = blk[dt, tp, di, pl.ds(tig * LANES, LANES)]
                    plsc.store_scatter(ob, [rvecs[tp][tig], dsp], v)

    read_blk(start, 0)
    read_blk(start + 2, 1)

    def pair_body(g, carry):
        for bi in (0, 1):
            i = 4 * g + 2 * bi
            wait_read(bi)

            @pl.when(i >= 4)
            def _():
                wait_write(bi, 256)

            transpose_blk(bi)
            write_ob(start + i, bi, 256)

            @pl.when(i + 4 < TPW)
            def _():
                read_blk(start + i + 4, bi)

        return carry

    lax.fori_loop(0, TPW // 4, pair_body, 0)
    wait_write(0, 256)
    wait_write(1, 256)

    @pl.when(wid <= 5)
    def _():
        tt = 32 * TPW + 2 * wid
        read_blk(tt, 0)
        wait_read(0)
        transpose_blk(0)
        write_ob(tt, 0, 256)
        wait_write(0, 256)

    @pl.when(wid == 6)
    def _():
        tt = TTB - 2
        nrows = VOCAB_SIZE - (TTB - 2) * 128
        read_blk(tt, 0)
        wait_read(0)
        transpose_blk(0)
        write_ob(tt, 0, nrows)
        wait_write(0, nrows)


@functools.partial(
    pl.kernel,
    mesh=_mesh,
    out_type=jax.ShapeDtypeStruct((MAXLEN * 8, NW, 8, BW), jnp.float32),
    scratch_types=[
        pltpu.VMEM((Lc, BW), jnp.int32),
        pltpu.VMEM((Lc, BW), jnp.int32),
        pltpu.VMEM((Lc * BW, EMBED_DIM), jnp.float32),
        pltpu.VMEM((Lc * BW, EMBED_DIM), jnp.float32),
        pltpu.VMEM((Sc * 8, 1, 8, BP), jnp.float32),
        pltpu.VMEM((Sc * 8, 1, 8, BP), jnp.float32),
        pltpu.VMEM((MAXLEN, EMBED_DIM), jnp.float32),
        pltpu.SemaphoreType.DMA,
        pltpu.SemaphoreType.DMA,
        pltpu.SemaphoreType.DMA,
        pltpu.SemaphoreType.DMA,
        pltpu.SemaphoreType.DMA,
        pltpu.SemaphoreType.DMA,
    ],
    compiler_params=pltpu.CompilerParams(
        use_tc_tiling_on_sc=False, needs_layout_passes=False),
)
def _embed(idxT_hbm, tok_hbm, pos_hbm, out_hbm,
           idx0, idx1, rows0, rows1, tout0, tout1, pos_v,
           i0, i1, g0, g1, s0, s1):
    wid = lax.axis_index("s") * NC + lax.axis_index("c")
    b0 = wid * BW

    idx_bufs = (idx0, idx1)
    rows_bufs = (rows0, rows1)
    tout_bufs = (tout0, tout1)
    isems = (i0, i1)
    gsems = (g0, g1)
    ssems = (s0, s1)

    pltpu.sync_copy(pos_hbm, pos_v)

    iota = lax.iota(jnp.int32, LANES)

    def stage_idx(c, bi):
        pltpu.async_copy(
            idxT_hbm.at[pl.ds(c * Lc, Lc), pl.ds(b0, BW)],
            idx_bufs[bi], isems[bi])

    def wait_idx(bi):
        pltpu.make_async_copy(
            idxT_hbm.at[pl.ds(0, Lc), pl.ds(0, BW)],
            idx_bufs[bi], isems[bi]).wait()

    def fire_gathers(bi):
        for l_loc in range(Lc):
            pltpu.async_copy(
                tok_hbm.at[idx_bufs[bi].at[l_loc]],
                rows_bufs[bi].at[pl.ds(l_loc * BW, BW)],
                gsems[bi])

    def wait_gathers(bi):
        pltpu.make_async_copy(
            tok_hbm.at[pl.ds(0, Lc * BW)], rows_bufs[bi], gsems[bi]).wait()

    def fire_scatter(c, h):
        pltpu.async_copy(
            tout_bufs[h].at[:, :, :, pl.ds(0, BW)],
            out_hbm.at[pl.ds((c * Lc + h * Sc) * 8, Sc * 8), pl.ds(wid, 1)],
            ssems[h])

    def wait_scatter(h):
        pltpu.make_async_copy(
            tout_bufs[h].at[:, :, :, pl.ds(0, BW)],
            out_hbm.at[pl.ds(0, Sc * 8), pl.ds(0, 1)], ssems[h]).wait()

    def transpose_add(c, bi, h):
        rows = rows_bufs[bi]
        tout = tout_bufs[h]
        for sl in range(Sc):
            l_loc = h * Sc + sl
            lrow = c * Lc + l_loc
            pos_q = [pos_v[lrow, pl.ds(q * LANES, LANES)] for q in range(4)]
            av = [((sl * 8 + 2 * q) + (iota >> 3)).astype(jnp.int32)
                  for q in range(4)]
            drv = iota & 7
            zv = jnp.zeros((LANES,), jnp.int32)

            @plsc.parallel_loop(0, BW, 1, unroll=4)
            def _bbody(b):
                row = l_loc * BW + b
                bsp = jnp.full((LANES,), b, jnp.int32)
                for q in range(4):
                    v = rows[row, pl.ds(q * LANES, LANES)] + pos_q[q]
                    plsc.store_scatter(tout, [av[q], zv, drv, bsp], v)

    def process(c, bi, first):
        for h in (0, 1):
            if not first:
                wait_scatter(h)
            transpose_add(c, bi, h)
            fire_scatter(c, h)

    stage_idx(0, 0)
    wait_idx(0)
    fire_gathers(0)
    stage_idx(1, 1)

    wait_idx(1)
    fire_gathers(1)
    wait_gathers(0)
    stage_idx(2, 0)
    process(0, 0, True)

    wait_idx(0)
    fire_gathers(0)
    wait_gathers(1)
    stage_idx(3, 1)
    process(1, 1, False)

    def pair_body(g, carry):
        for off in (0, 1):
            c = 2 * g + 2 + off
            bi = off
            nb = 1 - bi
            wait_idx(nb)
            fire_gathers(nb)
            wait_gathers(bi)
            stage_idx(c + 2, bi)
            process(c, bi, False)
        return carry

    lax.fori_loop(0, (NCH - 4) // 2, pair_body, 0)

    wait_idx(1)
    fire_gathers(1)
    wait_gathers(0)
    process(NCH - 2, 0, False)

    wait_gathers(1)
    process(NCH - 1, 1, False)

    wait_scatter(0)
    wait_scatter(1)


def kernel(inputs, token_table, pos_table):
    idxT = inputs.T.astype(jnp.int32)
    tok4 = jnp.pad(token_table.T, ((0, 0), (0, TTB * 128 - VOCAB_SIZE)))
    tok4 = tok4.reshape(8, 8, TTB, 128).transpose(0, 2, 1, 3)
    tokrm = _detile(tok4)
    out4 = _embed(idxT, tokrm, pos_table)
    t = out4.reshape(MAXLEN, 8, NW, 8, BW).transpose(0, 1, 3, 2, 4)
    return t.reshape(MAXLEN, EMBED_DIM, BATCH).transpose(2, 0, 1)
